# Initial kernel scaffold; baseline (speedup 1.0000x reference)
import jax
import jax.numpy as jnp
from jax.experimental import pallas as pl


def kernel(x_H, x_C, x_Others, ei_H_H, ei_H_C, ei_H_Others, ei_C_H, ei_C_C, ei_C_Others, ei_Others_H, ei_Others_C, ei_Others_Others, enc_W1, enc_b1, enc_W2, enc_b2, c1_Wr, c1_br, c1_Wt, c1_bt, c2_Wr, c2_br, c2_Wt, c2_bt, head_W, head_b):
    raise NotImplementedError("write your pallas kernel here")


# speedup vs baseline: 4.9379x; 4.9379x over previous
import functools

import jax
import jax.numpy as jnp
from jax import lax
from jax.experimental import pallas as pl
from jax.experimental.pallas import tpu as pltpu
from jax.experimental.pallas import tpu_sc as plsc

N_H, N_C, N_O = 100000, 50000, 10000
E = 320000
D_IN, D_H, D_O = 128, 32, 16

NC, NS, L = 2, 16, 16

P_H, P_C, P_O = 100352, 50176, 10240
PAD = (P_H, P_C, P_O)
RPT = (P_H // NS, P_C // NS, P_O // NS)
ZROWS = 256
ZCHUNKS = tuple([ZROWS] * (r // ZROWS) + ([r % ZROWS] if r % ZROWS else [])
                for r in RPT)

EROWS = E // 128
BLK = 32
NBLK = -(-EROWS // BLK)
TAIL = EROWS - (NBLK - 1) * BLK

REL_TYPES = ((0, 0), (0, 1), (0, 2), (1, 0), (1, 1), (1, 2),
             (2, 0), (2, 1), (2, 2))

_MM = functools.partial(jnp.dot, preferred_element_type=jnp.float32,
                        precision=lax.Precision.HIGHEST)


def _make_sc_conv(rel_ids, core0_ids):
    n_rel = len(rel_ids)
    core_rels = (tuple(r for r in rel_ids if r in core0_ids),
                 tuple(r for r in rel_ids if r not in core0_ids))
    out_type = tuple(jax.ShapeDtypeStruct((PAD[REL_TYPES[r][1]], D_O),
                                          jnp.float32) for r in rel_ids)

    mesh = plsc.VectorSubcoreMesh(core_axis_name="c", subcore_axis_name="s")

    @functools.partial(
        pl.kernel, out_type=out_type, mesh=mesh,
        compiler_params=pltpu.CompilerParams(use_tc_tiling_on_sc=False),
        scratch_types=[
            pltpu.VMEM_SHARED((P_H, D_O), jnp.float32),
            pltpu.VMEM((ZROWS, D_O), jnp.float32),
            pltpu.VMEM((BLK, 128), jnp.int32),
            pltpu.VMEM((BLK, 128), jnp.int32),
            pltpu.VMEM((128, D_O), jnp.float32),
            pltpu.SemaphoreType.DMA,
        ],
    )
    def conv(*refs):
        h_refs = refs[0:3]
        ei_refs = refs[3:3 + n_rel]
        z_ref = refs[3 + n_rel]
        o_refs = refs[4 + n_rel:4 + 2 * n_rel]
        agg_sh, zero_v, sidx_v, didx_v, rows_v, gsem = refs[4 + 2 * n_rel:]

        cid = lax.axis_index("c")
        sid = lax.axis_index("s")
        pltpu.sync_copy(z_ref, zero_v)

        def do_rel(pos, rel):
            src_t, dst_t = REL_TYPES[rel]
            rpt = RPT[dst_t]
            fr0 = sid * rpt
            off = 0
            for sz in ZCHUNKS[dst_t]:
                pltpu.sync_copy(zero_v.at[pl.ds(0, sz)],
                                agg_sh.at[pl.ds(fr0 + off, sz)])
                off += sz
            plsc.subcore_barrier()
            ei = ei_refs[pos]
            h_src = h_refs[src_t]

            def chunk(j, carry):
                pltpu.async_copy(h_src.at[sidx_v.at[j]], rows_v, gsem).wait()
                pltpu.sync_copy(rows_v, agg_sh.at[didx_v.at[j]], add=True)
                return carry

            def block(i, carry):
                k = sid + i * NS
                row0 = k * BLK

                @pl.when(k < NBLK - 1)
                def _():
                    pltpu.sync_copy(ei.at[0, pl.ds(row0, BLK)], sidx_v)
                    pltpu.sync_copy(ei.at[1, pl.ds(row0, BLK)], didx_v)
                    lax.fori_loop(0, BLK, chunk, 0)

                @pl.when(k == NBLK - 1)
                def _():
                    pltpu.sync_copy(ei.at[0, pl.ds(row0, TAIL)],
                                    sidx_v.at[pl.ds(0, TAIL)])
                    pltpu.sync_copy(ei.at[1, pl.ds(row0, TAIL)],
                                    didx_v.at[pl.ds(0, TAIL)])
                    lax.fori_loop(0, TAIL, chunk, 0)
                return carry

            lax.fori_loop(0, (NBLK - 1 - sid) // NS + 1, block, 0)

            plsc.subcore_barrier()
            pltpu.sync_copy(agg_sh.at[pl.ds(fr0, rpt)],
                            o_refs[pos].at[pl.ds(fr0, rpt)])

        for core in (0, 1):
            @pl.when(cid == core)
            def _(core=core):
                for rel in core_rels[core]:
                    do_rel(rel_ids.index(rel), rel)

    return conv


_SC_L1 = _make_sc_conv(tuple(range(9)), core0_ids=(0, 3, 6, 2))
_SC_L2 = _make_sc_conv((0, 3, 6, 1, 4, 7), core0_ids=(0, 3, 6))


def _encode(x, W1, b1, W2, b2, p_rows):
    n = x.shape[0]
    bn = 1000
    assert n % bn == 0

    def body(x_ref, w1_ref, b1_ref, w2_ref, b2_ref, o_ref):
        h = jnp.maximum(_MM(x_ref[...], w1_ref[...]) + b1_ref[...], 0.0)
        o_ref[...] = jnp.maximum(_MM(h, w2_ref[...]) + b2_ref[...], 0.0)

    return pl.pallas_call(
        body,
        grid=(n // bn,),
        in_specs=[
            pl.BlockSpec((bn, D_IN), lambda i: (i, 0)),
            pl.BlockSpec((D_IN, D_H), lambda i: (0, 0)),
            pl.BlockSpec((1, D_H), lambda i: (0, 0)),
            pl.BlockSpec((D_H, D_O), lambda i: (0, 0)),
            pl.BlockSpec((1, D_O), lambda i: (0, 0)),
        ],
        out_specs=pl.BlockSpec((bn, D_O), lambda i: (i, 0)),
        out_shape=jax.ShapeDtypeStruct((p_rows, D_O), jnp.float32),
    )(x, W1, b1.reshape(1, D_H), W2, b2.reshape(1, D_O))


def _combine(a0, a1, a2, h, W, b):
    p_rows = a0.shape[0]
    bn = 1024

    def body(a0_ref, a1_ref, a2_ref, h_ref, w_ref, b_ref, o_ref):
        x = jnp.concatenate(
            [a0_ref[...], a1_ref[...], a2_ref[...], h_ref[...]], axis=1)
        o_ref[...] = jnp.maximum(_MM(x, w_ref[...]) + b_ref[...], 0.0)

    blk = pl.BlockSpec((bn, D_O), lambda i: (i, 0))
    return pl.pallas_call(
        body,
        grid=(p_rows // bn,),
        in_specs=[blk, blk, blk, blk,
                  pl.BlockSpec((4 * D_O, D_O), lambda i: (0, 0)),
                  pl.BlockSpec((1, D_O), lambda i: (0, 0))],
        out_specs=blk,
        out_shape=jax.ShapeDtypeStruct((p_rows, D_O), jnp.float32),
    )(a0, a1, a2, h, W, b)


def _combine_head(a0, a1, a2, h, W, b, hw):
    p_rows = a0.shape[0]
    bn = 1024

    def body(a0_ref, a1_ref, a2_ref, h_ref, w_ref, b_ref, hw_ref, o_ref):
        x = jnp.concatenate(
            [a0_ref[...], a1_ref[...], a2_ref[...], h_ref[...]], axis=1)
        t = jnp.maximum(_MM(x, w_ref[...]) + b_ref[...], 0.0)
        o_ref[...] = jnp.sum(t * hw_ref[...], axis=1, keepdims=True)

    blk = pl.BlockSpec((bn, D_O), lambda i: (i, 0))
    return pl.pallas_call(
        body,
        grid=(p_rows // bn,),
        in_specs=[blk, blk, blk, blk,
                  pl.BlockSpec((4 * D_O, D_O), lambda i: (0, 0)),
                  pl.BlockSpec((1, D_O), lambda i: (0, 0)),
                  pl.BlockSpec((1, D_O), lambda i: (0, 0))],
        out_specs=pl.BlockSpec((bn, 1), lambda i: (i, 0)),
        out_shape=jax.ShapeDtypeStruct((p_rows, 1), jnp.float32),
    )(a0, a1, a2, h, W, b, hw)


def _conv_weights(Wr, br, Wt, bt, rels):
    r0, r1, r2 = rels
    W = jnp.concatenate([Wr[r0], Wr[r1], Wr[r2], Wt[r0] + Wt[r1] + Wt[r2]],
                        axis=0)
    b = (br[r0] + br[r1] + br[r2] + bt[r0] + bt[r1] + bt[r2]).reshape(1, D_O)
    return W, b


def kernel(x_H, x_C, x_Others, ei_H_H, ei_H_C, ei_H_Others, ei_C_H, ei_C_C,
           ei_C_Others, ei_Others_H, ei_Others_C, ei_Others_Others, enc_W1,
           enc_b1, enc_W2, enc_b2, c1_Wr, c1_br, c1_Wt, c1_bt, c2_Wr, c2_br,
           c2_Wt, c2_bt, head_W, head_b):
    hH = _encode(x_H, enc_W1[0], enc_b1[0], enc_W2[0], enc_b2[0], P_H)
    hC = _encode(x_C, enc_W1[1], enc_b1[1], enc_W2[1], enc_b2[1], P_C)
    hO = _encode(x_Others, enc_W1[2], enc_b1[2], enc_W2[2], enc_b2[2], P_O)

    eis = (ei_H_H, ei_H_C, ei_H_Others, ei_C_H, ei_C_C, ei_C_Others,
           ei_Others_H, ei_Others_C, ei_Others_Others)
    ei3 = [e.reshape(2, EROWS, 128) for e in eis]
    zblk = jnp.zeros((ZROWS, D_O), jnp.float32)

    a = _SC_L1(hH, hC, hO, *ei3, zblk)
    wH, bH = _conv_weights(c1_Wr, c1_br, c1_Wt, c1_bt, (0, 3, 6))
    wC, bC = _conv_weights(c1_Wr, c1_br, c1_Wt, c1_bt, (1, 4, 7))
    wO, bO = _conv_weights(c1_Wr, c1_br, c1_Wt, c1_bt, (2, 5, 8))
    h1H = _combine(a[0], a[3], a[6], hH, wH, bH)
    h1C = _combine(a[1], a[4], a[7], hC, wC, bC)
    h1O = _combine(a[2], a[5], a[8], hO, wO, bO)

    a2 = _SC_L2(h1H, h1C, h1O, ei3[0], ei3[3], ei3[6], ei3[1], ei3[4], ei3[7],
                zblk)
    w2H, b2H = _conv_weights(c2_Wr, c2_br, c2_Wt, c2_bt, (0, 3, 6))
    w2C, b2C = _conv_weights(c2_Wr, c2_br, c2_Wt, c2_bt, (1, 4, 7))
    pH = _combine_head(a2[0], a2[1], a2[2], h1H, w2H, b2H,
                       head_W[0].reshape(1, D_O))
    pC = _combine_head(a2[3], a2[4], a2[5], h1C, w2C, b2C,
                       head_W[1].reshape(1, D_O))

    out_H = pH[:N_H] + head_b[0]
    out_C = pC[:N_C] + head_b[1]
    return out_H, out_C


# speedup vs baseline: 9.0618x; 1.8351x over previous
import functools

import jax
import jax.numpy as jnp
from jax import lax
from jax.experimental import pallas as pl
from jax.experimental.pallas import tpu as pltpu
from jax.experimental.pallas import tpu_sc as plsc

N_H, N_C, N_O = 100000, 50000, 10000
E = 320000
D_IN, D_H, D_O = 128, 32, 16

NC, NS, L = 2, 16, 16

P_H, P_C, P_O = 100352, 50176, 10240
PAD = (P_H, P_C, P_O)
RPT = (P_H // NS, P_C // NS, P_O // NS)
ZROWS = 128
ZCHUNKS = tuple([ZROWS] * (r // ZROWS) + ([r % ZROWS] if r % ZROWS else [])
                for r in RPT)

EROWS = E // 128
BLK = 32
NBLK = -(-EROWS // BLK)
TAIL = EROWS - (NBLK - 1) * BLK
NBUF = 4
NBANK = 2

REL_TYPES = ((0, 0), (0, 1), (0, 2), (1, 0), (1, 1), (1, 2),
             (2, 0), (2, 1), (2, 2))

_MM = functools.partial(jnp.dot, preferred_element_type=jnp.float32)


def _make_sc_conv(rel_ids, core0_ids):
    n_rel = len(rel_ids)
    core_rels = (tuple(r for r in rel_ids if r in core0_ids),
                 tuple(r for r in rel_ids if r not in core0_ids))
    out_type = tuple(jax.ShapeDtypeStruct((PAD[REL_TYPES[r][1]], D_O),
                                          jnp.float32) for r in rel_ids)

    mesh = plsc.VectorSubcoreMesh(core_axis_name="c", subcore_axis_name="s")

    @functools.partial(
        pl.kernel, out_type=out_type, mesh=mesh,
        compiler_params=pltpu.CompilerParams(use_tc_tiling_on_sc=False),
        scratch_types=[
            pltpu.VMEM_SHARED((P_H, D_O), jnp.float32),
            pltpu.VMEM((ZROWS, D_O), jnp.float32),
            pltpu.VMEM((BLK, 128), jnp.int32),
            pltpu.VMEM((BLK, 128), jnp.int32),
        ] + [pltpu.VMEM((128, D_O), jnp.float32)
             for _ in range(NBANK * NBUF)]
          + [pltpu.SemaphoreType.DMA for _ in range(NBANK * NBUF)],
    )
    def conv(*refs):
        h_refs = refs[0:3]
        ei_refs = refs[3:3 + n_rel]
        z_ref = refs[3 + n_rel]
        o_refs = refs[4 + n_rel:4 + 2 * n_rel]
        scratch = refs[4 + 2 * n_rel:]
        agg_sh, zero_v, sidx_v, didx_v = scratch[:4]
        nb = NBANK * NBUF
        rows_v = scratch[4:4 + nb]
        gsem = scratch[4 + nb:4 + 2 * nb]

        cid = lax.axis_index("c")
        sid = lax.axis_index("s")
        pltpu.sync_copy(z_ref, zero_v)

        def do_rel(pos, rel):
            src_t, dst_t = REL_TYPES[rel]
            rpt = RPT[dst_t]
            fr0 = sid * rpt
            plsc.subcore_barrier()
            off = 0
            for sz in ZCHUNKS[dst_t]:
                pltpu.sync_copy(zero_v.at[pl.ds(0, sz)],
                                agg_sh.at[pl.ds(fr0 + off, sz)])
                off += sz
            plsc.subcore_barrier()
            ei = ei_refs[pos]
            h_src = h_refs[src_t]

            def block(i, carry):
                k = sid + i * NS
                row0 = k * BLK

                NGRP = BLK // NBUF

                def fire(g, bank):
                    for b in range(NBUF):
                        pltpu.async_copy(
                            h_src.at[sidx_v.at[g * NBUF + b]],
                            rows_v[bank * NBUF + b], gsem[bank * NBUF + b])

                def drain_scatter(g, bank):
                    for b in range(NBUF):
                        pltpu.make_async_copy(
                            h_src.at[sidx_v.at[g * NBUF + b]],
                            rows_v[bank * NBUF + b],
                            gsem[bank * NBUF + b]).wait()
                        pltpu.sync_copy(rows_v[bank * NBUF + b],
                                        agg_sh.at[didx_v.at[g * NBUF + b]],
                                        add=True)

                @pl.when(k < NBLK - 1)
                def _():
                    pltpu.sync_copy(ei.at[0, pl.ds(row0, BLK)], sidx_v)
                    pltpu.sync_copy(ei.at[1, pl.ds(row0, BLK)], didx_v)

                    fire(0, 0)

                    def pair(p, c):
                        fire(2 * p + 1, 1)
                        drain_scatter(2 * p, 0)

                        @pl.when(p < NGRP // 2 - 1)
                        def _():
                            fire(2 * p + 2, 0)

                        drain_scatter(2 * p + 1, 1)
                        return c

                    lax.fori_loop(0, NGRP // 2, pair, 0)

                @pl.when(k == NBLK - 1)
                def _():
                    pltpu.sync_copy(ei.at[0, pl.ds(row0, TAIL)],
                                    sidx_v.at[pl.ds(0, TAIL)])
                    pltpu.sync_copy(ei.at[1, pl.ds(row0, TAIL)],
                                    didx_v.at[pl.ds(0, TAIL)])

                    def chunk(j, c):
                        pltpu.async_copy(h_src.at[sidx_v.at[j]], rows_v[0],
                                         gsem[0]).wait()
                        pltpu.sync_copy(rows_v[0], agg_sh.at[didx_v.at[j]],
                                        add=True)
                        return c

                    lax.fori_loop(0, TAIL, chunk, 0)
                return carry

            lax.fori_loop(0, (NBLK - 1 - sid) // NS + 1, block, 0)

            plsc.subcore_barrier()
            pltpu.sync_copy(agg_sh.at[pl.ds(fr0, rpt)],
                            o_refs[pos].at[pl.ds(fr0, rpt)])

        for core in (0, 1):
            @pl.when(cid == core)
            def _(core=core):
                for rel in core_rels[core]:
                    do_rel(rel_ids.index(rel), rel)

    return conv


_SC_L1 = _make_sc_conv(tuple(range(9)), core0_ids=(0, 3, 6, 2))
_SC_L2 = _make_sc_conv((0, 3, 6, 1, 4, 7), core0_ids=(0, 3, 6))


def _encode(x, W1, b1, W2, b2, p_rows):
    n = x.shape[0]
    bn = 1000
    assert n % bn == 0

    def body(x_ref, w1_ref, b1_ref, w2_ref, b2_ref, o_ref):
        h = jnp.maximum(_MM(x_ref[...], w1_ref[...]) + b1_ref[...], 0.0)
        o_ref[...] = jnp.maximum(_MM(h, w2_ref[...]) + b2_ref[...], 0.0)

    return pl.pallas_call(
        body,
        grid=(n // bn,),
        in_specs=[
            pl.BlockSpec((bn, D_IN), lambda i: (i, 0)),
            pl.BlockSpec((D_IN, D_H), lambda i: (0, 0)),
            pl.BlockSpec((1, D_H), lambda i: (0, 0)),
            pl.BlockSpec((D_H, D_O), lambda i: (0, 0)),
            pl.BlockSpec((1, D_O), lambda i: (0, 0)),
        ],
        out_specs=pl.BlockSpec((bn, D_O), lambda i: (i, 0)),
        out_shape=jax.ShapeDtypeStruct((p_rows, D_O), jnp.float32),
    )(x, W1, b1.reshape(1, D_H), W2, b2.reshape(1, D_O))


def _gconv_sum(aggs, hh, w, b):
    acc = None
    for i in range(3):
        g = ((_MM(aggs[i], w[i * D_O:(i + 1) * D_O])
              + b[:, i * 2 * D_O:i * 2 * D_O + D_O])
             + _MM(hh, w[(3 + i) * D_O:(4 + i) * D_O])
             ) + b[:, i * 2 * D_O + D_O:(i + 1) * 2 * D_O]
        acc = g if acc is None else acc + g
    return acc


def _combine(a0, a1, a2, h, W, b):
    p_rows = a0.shape[0]
    bn = 1024

    def body(a0_ref, a1_ref, a2_ref, h_ref, w_ref, b_ref, o_ref):
        o_ref[...] = jnp.maximum(
            _gconv_sum([a0_ref[...], a1_ref[...], a2_ref[...]], h_ref[...],
                       w_ref[...], b_ref[...]), 0.0)

    blk = pl.BlockSpec((bn, D_O), lambda i: (i, 0))
    return pl.pallas_call(
        body,
        grid=(p_rows // bn,),
        in_specs=[blk, blk, blk, blk,
                  pl.BlockSpec((6 * D_O, D_O), lambda i: (0, 0)),
                  pl.BlockSpec((1, 6 * D_O), lambda i: (0, 0))],
        out_specs=blk,
        out_shape=jax.ShapeDtypeStruct((p_rows, D_O), jnp.float32),
    )(a0, a1, a2, h, W, b)


def _combine_head(a0, a1, a2, h, W, b, hw):
    p_rows = a0.shape[0]
    bn = 1024

    def body(a0_ref, a1_ref, a2_ref, h_ref, w_ref, b_ref, hw_ref, o_ref):
        t = jnp.maximum(
            _gconv_sum([a0_ref[...], a1_ref[...], a2_ref[...]], h_ref[...],
                       w_ref[...], b_ref[...]), 0.0)
        tb = t.astype(jnp.bfloat16).astype(jnp.float32)
        hwb = hw_ref[...].astype(jnp.bfloat16).astype(jnp.float32)
        o_ref[...] = jnp.sum(tb * hwb, axis=1, keepdims=True)

    blk = pl.BlockSpec((bn, D_O), lambda i: (i, 0))
    return pl.pallas_call(
        body,
        grid=(p_rows // bn,),
        in_specs=[blk, blk, blk, blk,
                  pl.BlockSpec((6 * D_O, D_O), lambda i: (0, 0)),
                  pl.BlockSpec((1, 6 * D_O), lambda i: (0, 0)),
                  pl.BlockSpec((1, D_O), lambda i: (0, 0))],
        out_specs=pl.BlockSpec((bn, 1), lambda i: (i, 0)),
        out_shape=jax.ShapeDtypeStruct((p_rows, 1), jnp.float32),
    )(a0, a1, a2, h, W, b, hw)


def _conv_weights(Wr, br, Wt, bt, rels):
    r0, r1, r2 = rels
    W = jnp.concatenate([Wr[r0], Wr[r1], Wr[r2], Wt[r0], Wt[r1], Wt[r2]],
                        axis=0)
    b = jnp.concatenate([br[r0], bt[r0], br[r1], bt[r1], br[r2], bt[r2]],
                        axis=0).reshape(1, 6 * D_O)
    return W, b


def kernel(x_H, x_C, x_Others, ei_H_H, ei_H_C, ei_H_Others, ei_C_H, ei_C_C,
           ei_C_Others, ei_Others_H, ei_Others_C, ei_Others_Others, enc_W1,
           enc_b1, enc_W2, enc_b2, c1_Wr, c1_br, c1_Wt, c1_bt, c2_Wr, c2_br,
           c2_Wt, c2_bt, head_W, head_b):
    hH = _encode(x_H, enc_W1[0], enc_b1[0], enc_W2[0], enc_b2[0], P_H)
    hC = _encode(x_C, enc_W1[1], enc_b1[1], enc_W2[1], enc_b2[1], P_C)
    hO = _encode(x_Others, enc_W1[2], enc_b1[2], enc_W2[2], enc_b2[2], P_O)

    eis = (ei_H_H, ei_H_C, ei_H_Others, ei_C_H, ei_C_C, ei_C_Others,
           ei_Others_H, ei_Others_C, ei_Others_Others)
    ei3 = [e.reshape(2, EROWS, 128) for e in eis]
    zblk = jnp.zeros((ZROWS, D_O), jnp.float32)

    a = _SC_L1(hH, hC, hO, *ei3, zblk)
    wH, bH = _conv_weights(c1_Wr, c1_br, c1_Wt, c1_bt, (0, 3, 6))
    wC, bC = _conv_weights(c1_Wr, c1_br, c1_Wt, c1_bt, (1, 4, 7))
    wO, bO = _conv_weights(c1_Wr, c1_br, c1_Wt, c1_bt, (2, 5, 8))
    h1H = _combine(a[0], a[3], a[6], hH, wH, bH)
    h1C = _combine(a[1], a[4], a[7], hC, wC, bC)
    h1O = _combine(a[2], a[5], a[8], hO, wO, bO)

    a2 = _SC_L2(h1H, h1C, h1O, ei3[0], ei3[3], ei3[6], ei3[1], ei3[4], ei3[7],
                zblk)
    w2H, b2H = _conv_weights(c2_Wr, c2_br, c2_Wt, c2_bt, (0, 3, 6))
    w2C, b2C = _conv_weights(c2_Wr, c2_br, c2_Wt, c2_bt, (1, 4, 7))
    pH = _combine_head(a2[0], a2[1], a2[2], h1H, w2H, b2H,
                       head_W[0].reshape(1, D_O))
    pC = _combine_head(a2[3], a2[4], a2[5], h1C, w2C, b2C,
                       head_W[1].reshape(1, D_O))

    out_H = pH[:N_H] + head_b[0]
    out_C = pC[:N_C] + head_b[1]
    return out_H, out_C
